# baseline (speedup 1.0000x reference)
import functools

import jax
import jax.numpy as jnp
import numpy as np
from jax.experimental import pallas as pl
from jax.experimental.pallas import tpu as pltpu

_ALPHA = np.float32(3.0)
_ONE_M_ALPHA = np.float32(1.0) - _ALPHA

_BLOCK_P = 5000


def _host_uniform_table(n_rows, width):
    n = n_rows * width

    def rotl(x, d):
        return ((x << np.uint32(d)) | (x >> np.uint32(32 - d))).astype(np.uint32)

    ks = [np.uint32(0), np.uint32(42), np.uint32(0 ^ 42 ^ 0x1BD11BDA)]
    rot0 = (13, 15, 26, 6)
    rot1 = (17, 29, 16, 24)
    x0 = np.full(n, ks[0], dtype=np.uint32)
    x1 = (np.arange(n, dtype=np.uint32) + ks[1]).astype(np.uint32)
    for i in range(5):
        for r in (rot0 if i % 2 == 0 else rot1):
            x0 = (x0 + x1).astype(np.uint32)
            x1 = rotl(x1, r)
            x1 = (x1 ^ x0).astype(np.uint32)
        x0 = (x0 + ks[(i + 1) % 3]).astype(np.uint32)
        x1 = (x1 + ks[(i + 2) % 3] + np.uint32(i + 1)).astype(np.uint32)
    bits = x0 ^ x1

    tiny = np.float32(np.finfo(np.float32).tiny)
    f = ((bits >> np.uint32(9)) | np.uint32(0x3F800000)).view(np.float32) \
        - np.float32(1.0)
    u = np.maximum(tiny, f * (np.float32(1.0) - tiny) + tiny)
    u = u.reshape(n_rows, width)
    packed = np.empty((width // 2, 2 * n_rows), dtype=np.float32)
    packed[:, :n_rows] = u[:, 0::2].T
    packed[:, n_rows:] = u[:, 1::2].T
    return packed


def _sample_block(lt_ref, unif_ref, out_ref, m_run, i_run, iota_s,
                  *, n, n_steps):
    pid = pl.program_id(0)
    n2 = 2 * n

    @pl.when(pid == 0)
    def _init():
        m_run[...] = jnp.full((1, n2), -jnp.inf, jnp.float32)
        i_run[...] = jnp.zeros((1, n2), jnp.int32)
        lane = jax.lax.broadcasted_iota(jnp.int32, (_BLOCK_P, n2), 1)
        row2 = jax.lax.broadcasted_iota(jnp.int32, (_BLOCK_P, n2), 0) * 2
        iota_s[...] = row2 + jnp.where(lane < n, 0, 1)

    x = lt_ref[...]
    u = jnp.concatenate([x[:, 0:n], x[:, n2:n2 + n]], axis=1)
    c = jnp.concatenate([x[:, n:n2], x[:, n2 + n:]], axis=1)
    cfg = _ONE_M_ALPHA * u + _ALPHA * c
    g = -jnp.log(-jnp.log(unif_ref[...]))
    val = cfg + g

    m_blk = jnp.max(val, axis=0, keepdims=True)
    i_loc = jnp.min(jnp.where(val == m_blk, iota_s[...], jnp.int32(0x7FFFFFFF)),
                    axis=0, keepdims=True)
    i_blk = i_loc + pid * (2 * _BLOCK_P)

    upd = m_blk > m_run[...]
    m_run[...] = jnp.where(upd, m_blk, m_run[...])
    i_run[...] = jnp.where(upd, i_blk, i_run[...])

    @pl.when(pid == n_steps - 1)
    def _emit():
        m_e = m_run[:, :n]
        m_o = m_run[:, n:]
        i_e = i_run[:, :n]
        i_o = i_run[:, n:]
        pick_e = (m_e > m_o) | ((m_e == m_o) & (i_e < i_o))
        out_ref[...] = jnp.where(pick_e, i_e, i_o)


def kernel(logits, start, end, memo):
    shape = logits.shape
    width = shape[-1]
    flat = logits.reshape(-1, width)
    n = flat.shape[0] // 2
    n_steps = (width // 2) // _BLOCK_P

    lpairs = flat.T.reshape(width // 2, 4 * n)
    unif = jnp.asarray(_host_uniform_table(n, width))

    tokens = pl.pallas_call(
        functools.partial(_sample_block, n=n, n_steps=n_steps),
        grid=(n_steps,),
        in_specs=[
            pl.BlockSpec((_BLOCK_P, 4 * n), lambda i: (i, 0)),
            pl.BlockSpec((_BLOCK_P, 2 * n), lambda i: (i, 0)),
        ],
        out_specs=pl.BlockSpec((1, n), lambda i: (0, 0)),
        out_shape=jax.ShapeDtypeStruct((1, n), jnp.int32),
        scratch_shapes=[
            pltpu.VMEM((1, 2 * n), jnp.float32),
            pltpu.VMEM((1, 2 * n), jnp.int32),
            pltpu.VMEM((_BLOCK_P, 2 * n), jnp.int32),
        ],
    )(lpairs, unif)

    tokens = tokens.reshape(n)
    tokens = jnp.concatenate([tokens, tokens], axis=0)
    tokens = tokens + start + (end - width)
    return tokens.reshape(shape[:-1])


# speedup vs baseline: 2.6031x; 2.6031x over previous
import functools

import jax
import jax.numpy as jnp
import numpy as np
from jax.experimental import pallas as pl
from jax.experimental.pallas import tpu as pltpu

_ALPHA = np.float32(3.0)
_ONE_M_ALPHA = np.float32(1.0) - _ALPHA

_BLOCK_P = 5000


def _host_uniform_table(n_rows, width):
    n = n_rows * width

    def rotl(x, d):
        return ((x << np.uint32(d)) | (x >> np.uint32(32 - d))).astype(np.uint32)

    ks = [np.uint32(0), np.uint32(42), np.uint32(0 ^ 42 ^ 0x1BD11BDA)]
    rot0 = (13, 15, 26, 6)
    rot1 = (17, 29, 16, 24)
    x0 = np.full(n, ks[0], dtype=np.uint32)
    x1 = (np.arange(n, dtype=np.uint32) + ks[1]).astype(np.uint32)
    for i in range(5):
        for r in (rot0 if i % 2 == 0 else rot1):
            x0 = (x0 + x1).astype(np.uint32)
            x1 = rotl(x1, r)
            x1 = (x1 ^ x0).astype(np.uint32)
        x0 = (x0 + ks[(i + 1) % 3]).astype(np.uint32)
        x1 = (x1 + ks[(i + 2) % 3] + np.uint32(i + 1)).astype(np.uint32)
    bits = x0 ^ x1

    tiny = np.float32(np.finfo(np.float32).tiny)
    f = ((bits >> np.uint32(9)) | np.uint32(0x3F800000)).view(np.float32) \
        - np.float32(1.0)
    u = np.maximum(tiny, f * (np.float32(1.0) - tiny) + tiny)
    u = u.reshape(n_rows, width)
    half = width // 2
    packed = np.empty((half, 2 * n_rows), dtype=np.float32)
    packed[:, :n_rows] = u[:, :half].T
    packed[:, n_rows:] = u[:, half:].T
    return packed


def _sample_block(la_ref, lb_ref, unif_ref, out_ref, m_run, i_run, iota_s,
                  *, n, n_steps, half):
    pid = pl.program_id(0)
    n2 = 2 * n

    @pl.when(pid == 0)
    def _init():
        m_run[...] = jnp.full((1, n2), -jnp.inf, jnp.float32)
        i_run[...] = jnp.zeros((1, n2), jnp.int32)
        lane = jax.lax.broadcasted_iota(jnp.int32, (_BLOCK_P, n2), 1)
        row = jax.lax.broadcasted_iota(jnp.int32, (_BLOCK_P, n2), 0)
        iota_s[...] = row + jnp.where(lane < n, 0, half)

    xa = la_ref[...]
    xb = lb_ref[...]
    u = jnp.concatenate([xa[:, :n], xb[:, :n]], axis=1)
    c = jnp.concatenate([xa[:, n:], xb[:, n:]], axis=1)
    cfg = _ONE_M_ALPHA * u + _ALPHA * c
    g = -jnp.log(-jnp.log(unif_ref[...]))
    val = cfg + g

    m_blk = jnp.max(val, axis=0, keepdims=True)
    i_loc = jnp.min(jnp.where(val == m_blk, iota_s[...], jnp.int32(0x7FFFFFFF)),
                    axis=0, keepdims=True)
    i_blk = i_loc + pid * _BLOCK_P

    upd = m_blk > m_run[...]
    m_run[...] = jnp.where(upd, m_blk, m_run[...])
    i_run[...] = jnp.where(upd, i_blk, i_run[...])

    @pl.when(pid == n_steps - 1)
    def _emit():
        m_lo = m_run[:, :n]
        m_hi = m_run[:, n:]
        i_lo = i_run[:, :n]
        i_hi = i_run[:, n:]
        out_ref[...] = jnp.where(m_lo >= m_hi, i_lo, i_hi)


def kernel(logits, start, end, memo):
    shape = logits.shape
    width = shape[-1]
    flat = logits.reshape(-1, width)
    n = flat.shape[0] // 2
    half = width // 2
    n_steps = half // _BLOCK_P

    ltrans = flat.T
    unif = jnp.asarray(_host_uniform_table(n, width))

    tokens = pl.pallas_call(
        functools.partial(_sample_block, n=n, n_steps=n_steps, half=half),
        grid=(n_steps,),
        in_specs=[
            pl.BlockSpec((_BLOCK_P, 2 * n), lambda i: (i, 0)),
            pl.BlockSpec((_BLOCK_P, 2 * n), lambda i, _ns=n_steps: (i + _ns, 0)),
            pl.BlockSpec((_BLOCK_P, 2 * n), lambda i: (i, 0)),
        ],
        out_specs=pl.BlockSpec((1, n), lambda i: (0, 0)),
        out_shape=jax.ShapeDtypeStruct((1, n), jnp.int32),
        scratch_shapes=[
            pltpu.VMEM((1, 2 * n), jnp.float32),
            pltpu.VMEM((1, 2 * n), jnp.int32),
            pltpu.VMEM((_BLOCK_P, 2 * n), jnp.int32),
        ],
    )(ltrans, ltrans, unif)

    tokens = tokens.reshape(n)
    tokens = jnp.concatenate([tokens, tokens], axis=0)
    tokens = tokens + start + (end - width)
    return tokens.reshape(shape[:-1])
